# baseline (speedup 1.0000x reference)
import jax
import jax.numpy as jnp
from jax import lax
from jax.experimental import pallas as pl
from jax.experimental.pallas import tpu as pltpu
from jax.experimental.pallas import tpu_sc as plsc

N = 10000
E = 320000
D = 128
N_PAD = 10240
ACC_ROWS = 10000
D_AUG = 144
NW = 32
KA = 64
KB = 128
SEGA1, SEGB1 = 118, 20
SEGA2, SEGB2 = 126, 16
STRIPE = ACC_ROWS // 16
BLK = 512


def _make_agg(d, nbuf, sega, segb):
    mesh = plsc.VectorSubcoreMesh(core_axis_name="c", subcore_axis_name="s")

    def body(x_hbm, idxa_hbm, idxb_hbm, z_hbm, out_hbm,
             srcva, dstva, srcvb, dstvb, buf, sems, acc):
        c = lax.axis_index("c")
        s = lax.axis_index("s")
        pltpu.sync_copy(z_hbm, acc.at[pl.ds(s * STRIPE, STRIPE)])
        plsc.subcore_barrier()

        bufs = [buf.at[pl.ds(b * KA, KA)] for b in range(nbuf)]

        @pl.when(c == 0)
        def _sc0():
            for seg in range(2):
                pltpu.sync_copy(idxa_hbm.at[s, seg, 0], srcva)
                pltpu.sync_copy(idxa_hbm.at[s, seg, 1], dstva)
                for b in range(nbuf):
                    pltpu.async_copy(x_hbm.at[srcva.at[b]], bufs[b], sems[b])

                def group(g, carry):
                    for b in range(nbuf):
                        j = g * nbuf + b
                        pltpu.make_async_copy(x_hbm.at[srcva.at[j]], bufs[b],
                                              sems[b]).wait()
                        pltpu.sync_copy(bufs[b], acc.at[dstva.at[j]], add=True)
                        pltpu.async_copy(x_hbm.at[srcva.at[j + nbuf]],
                                         bufs[b], sems[b])
                    return carry

                lax.fori_loop(0, sega // nbuf - 1, group, 0)
                for b in range(nbuf):
                    j = sega - nbuf + b
                    pltpu.make_async_copy(x_hbm.at[srcva.at[j]], bufs[b],
                                          sems[b]).wait()
                    pltpu.sync_copy(bufs[b], acc.at[dstva.at[j]], add=True)

        @pl.when(c == 1)
        def _sc1():
            sbuf = buf.at[pl.ds(0, KB)]
            for seg in range(2):
                pltpu.sync_copy(idxb_hbm.at[s, seg, 0], srcvb)
                pltpu.sync_copy(idxb_hbm.at[s, seg, 1], dstvb)

                def step(j, carry):
                    pltpu.sync_copy(x_hbm.at[srcvb.at[j]], sbuf)
                    pltpu.sync_copy(sbuf, acc.at[dstvb.at[j]], add=True)
                    return carry

                lax.fori_loop(0, segb, step, 0)

        plsc.subcore_barrier()
        pltpu.sync_copy(acc.at[pl.ds(s * STRIPE, STRIPE)],
                        out_hbm.at[pl.ds(c * N_PAD + s * STRIPE, STRIPE)])

    return pl.kernel(
        body,
        out_type=jax.ShapeDtypeStruct((2 * N_PAD, d), jnp.float32),
        mesh=mesh,
        compiler_params=pltpu.CompilerParams(use_tc_tiling_on_sc=False),
        scratch_types=[
            pltpu.VMEM((sega, KA), jnp.int32),
            pltpu.VMEM((sega, KA), jnp.int32),
            pltpu.VMEM((segb, KB), jnp.int32),
            pltpu.VMEM((segb, KB), jnp.int32),
            pltpu.VMEM((nbuf * KA if nbuf * KA > KB else KB, d), jnp.float32),
            [pltpu.SemaphoreType.DMA for _ in range(nbuf)],
            pltpu.VMEM_SHARED((ACC_ROWS, d), jnp.float32),
        ],
    )


_agg_aug = _make_agg(D_AUG, 2, SEGA1, SEGB1)
_agg_plain = _make_agg(D, 3, SEGA2, SEGB2)


def _dense1_body(p0, p1, x_ref, wl, wr, b, h_ref, inv_ref):
    s = p0[...] + p1[...]
    deg = s[:, D:D + 1]
    inv = 1.0 / jnp.maximum(deg, 1.0)
    mean = s[:, :D] * inv
    h = (jnp.dot(mean, wl[...], preferred_element_type=jnp.float32)
         + jnp.dot(x_ref[...], wr[...], preferred_element_type=jnp.float32)
         + b[...])
    row = pl.program_id(0) * BLK + lax.broadcasted_iota(jnp.int32, (BLK, 1), 0)
    h_ref[...] = jnp.where(row < N, jnp.maximum(h, 0.0), 0.0)
    inv_ref[...] = inv


_dense1 = pl.pallas_call(
    _dense1_body,
    grid=(N_PAD // BLK,),
    in_specs=[
        pl.BlockSpec((BLK, D_AUG), lambda i: (i, 0)),
        pl.BlockSpec((BLK, D_AUG), lambda i: (i, 0)),
        pl.BlockSpec((BLK, D), lambda i: (i, 0)),
        pl.BlockSpec((D, D), lambda i: (0, 0)),
        pl.BlockSpec((D, D), lambda i: (0, 0)),
        pl.BlockSpec((1, D), lambda i: (0, 0)),
    ],
    out_specs=[pl.BlockSpec((BLK, D), lambda i: (i, 0)),
               pl.BlockSpec((BLK, 1), lambda i: (i, 0))],
    out_shape=[jax.ShapeDtypeStruct((N_PAD, D), jnp.float32),
               jax.ShapeDtypeStruct((N_PAD, 1), jnp.float32)],
)


def _dense2_body(p0, p1, h_ref, inv_ref, wl, wr, b, out_ref):
    mean = (p0[...] + p1[...]) * inv_ref[...]
    out_ref[...] = (jnp.dot(mean, wl[...], preferred_element_type=jnp.float32)
                    + jnp.dot(h_ref[...], wr[...], preferred_element_type=jnp.float32)
                    + b[...])


_dense2 = pl.pallas_call(
    _dense2_body,
    grid=(N_PAD // BLK,),
    in_specs=[
        pl.BlockSpec((BLK, D), lambda i: (i, 0)),
        pl.BlockSpec((BLK, D), lambda i: (i, 0)),
        pl.BlockSpec((BLK, D), lambda i: (i, 0)),
        pl.BlockSpec((BLK, 1), lambda i: (i, 0)),
        pl.BlockSpec((D, D), lambda i: (0, 0)),
        pl.BlockSpec((D, D), lambda i: (0, 0)),
        pl.BlockSpec((1, D), lambda i: (0, 0)),
    ],
    out_specs=pl.BlockSpec((BLK, D), lambda i: (i, 0)),
    out_shape=jax.ShapeDtypeStruct((N_PAD, D), jnp.float32),
)


def kernel(x, edge_index, W1l, b1l, W1r, W2l, b2l, W2r):
    src = edge_index[0].astype(jnp.int32)
    dst = edge_index[1].astype(jnp.int32)

    def split_idx(sega, segb):
        ea = 16 * 2 * sega * KA
        e_pad = ea + 16 * 2 * segb * KB
        pad_dst = jnp.arange(e_pad - E, dtype=jnp.int32) % N
        src_p = jnp.concatenate([src, jnp.full((e_pad - E,), N, jnp.int32)])
        dst_p = jnp.concatenate([dst, pad_dst])
        idxa = jnp.stack([src_p[:ea].reshape(16, 2, sega, KA),
                          dst_p[:ea].reshape(16, 2, sega, KA)], axis=2)
        idxb = jnp.stack([src_p[ea:].reshape(16, 2, segb, KB),
                          dst_p[ea:].reshape(16, 2, segb, KB)], axis=2)
        return idxa, idxb

    idxa1, idxb1 = split_idx(SEGA1, SEGB1)
    idxa2, idxb2 = split_idx(SEGA2, SEGB2)

    x_aug = jnp.zeros((N_PAD, D_AUG), jnp.float32)
    x_aug = x_aug.at[:N, :D].set(x)
    x_aug = x_aug.at[:N, D].set(1.0)
    x_pad = x_aug[:, :D]
    z_aug = jnp.zeros((STRIPE, D_AUG), jnp.float32)
    z_plain = jnp.zeros((STRIPE, D), jnp.float32)

    p = _agg_aug(x_aug, idxa1, idxb1, z_aug)
    h, inv = _dense1(p[:N_PAD], p[N_PAD:], x_pad, W1l.T, W1r.T, b1l[None, :])
    p2 = _agg_plain(h, idxa2, idxb2, z_plain)
    out = _dense2(p2[:N_PAD], p2[N_PAD:], h, inv, W2l.T, W2r.T, b2l[None, :])
    return out[:N]


# speedup vs baseline: 1.0072x; 1.0072x over previous
import jax
import jax.numpy as jnp
from jax import lax
from jax.experimental import pallas as pl
from jax.experimental.pallas import tpu as pltpu
from jax.experimental.pallas import tpu_sc as plsc

N = 10000
E = 320000
D = 128
N_PAD = 10240
ACC_ROWS = 10000
D_AUG = 144
NW = 32
KA = 64
KB = 128
SEGA1, SEGB1 = 108, 25
SEGA2, SEGB2 = 108, 25
STRIPE = ACC_ROWS // 16
BLK = 512


def _make_agg(d, nbuf, sega, segb):
    mesh = plsc.VectorSubcoreMesh(core_axis_name="c", subcore_axis_name="s")

    def body(x_hbm, idxa_hbm, idxb_hbm, z_hbm, out_hbm,
             srcva, dstva, srcvb, dstvb, buf, sems, acc):
        c = lax.axis_index("c")
        s = lax.axis_index("s")
        pltpu.sync_copy(z_hbm, acc.at[pl.ds(s * STRIPE, STRIPE)])
        plsc.subcore_barrier()

        bufs = [buf.at[pl.ds(b * KA, KA)] for b in range(nbuf)]

        @pl.when(c == 0)
        def _sc0():
            for seg in range(2):
                pltpu.sync_copy(idxa_hbm.at[s, seg, 0], srcva)
                pltpu.sync_copy(idxa_hbm.at[s, seg, 1], dstva)
                for b in range(nbuf):
                    pltpu.async_copy(x_hbm.at[srcva.at[b]], bufs[b], sems[b])

                def group(g, carry):
                    for b in range(nbuf):
                        j = g * nbuf + b
                        pltpu.make_async_copy(x_hbm.at[srcva.at[j]], bufs[b],
                                              sems[b]).wait()
                        pltpu.sync_copy(bufs[b], acc.at[dstva.at[j]], add=True)
                        pltpu.async_copy(x_hbm.at[srcva.at[j + nbuf]],
                                         bufs[b], sems[b])
                    return carry

                lax.fori_loop(0, sega // nbuf - 1, group, 0)
                for b in range(nbuf):
                    j = sega - nbuf + b
                    pltpu.make_async_copy(x_hbm.at[srcva.at[j]], bufs[b],
                                          sems[b]).wait()
                    pltpu.sync_copy(bufs[b], acc.at[dstva.at[j]], add=True)

        @pl.when(c == 1)
        def _sc1():
            sbuf = buf.at[pl.ds(0, KB)]
            for seg in range(2):
                pltpu.sync_copy(idxb_hbm.at[s, seg, 0], srcvb)
                pltpu.sync_copy(idxb_hbm.at[s, seg, 1], dstvb)

                def step(j, carry):
                    pltpu.sync_copy(x_hbm.at[srcvb.at[j]], sbuf)
                    pltpu.sync_copy(sbuf, acc.at[dstvb.at[j]], add=True)
                    return carry

                lax.fori_loop(0, segb, step, 0)

        plsc.subcore_barrier()
        pltpu.sync_copy(acc.at[pl.ds(s * STRIPE, STRIPE)],
                        out_hbm.at[pl.ds(c * N_PAD + s * STRIPE, STRIPE)])

    return pl.kernel(
        body,
        out_type=jax.ShapeDtypeStruct((2 * N_PAD, d), jnp.float32),
        mesh=mesh,
        compiler_params=pltpu.CompilerParams(use_tc_tiling_on_sc=False),
        scratch_types=[
            pltpu.VMEM((sega, KA), jnp.int32),
            pltpu.VMEM((sega, KA), jnp.int32),
            pltpu.VMEM((segb, KB), jnp.int32),
            pltpu.VMEM((segb, KB), jnp.int32),
            pltpu.VMEM((nbuf * KA if nbuf * KA > KB else KB, d), jnp.float32),
            [pltpu.SemaphoreType.DMA for _ in range(nbuf)],
            pltpu.VMEM_SHARED((ACC_ROWS, d), jnp.float32),
        ],
    )


_agg_aug = _make_agg(D_AUG, 2, SEGA1, SEGB1)
_agg_plain = _make_agg(D, 3, SEGA2, SEGB2)


def _dense1_body(p0, p1, x_ref, wl, wr, b, h_ref, inv_ref):
    s = p0[...] + p1[...]
    deg = s[:, D:D + 1]
    inv = 1.0 / jnp.maximum(deg, 1.0)
    mean = s[:, :D] * inv
    h = (jnp.dot(mean, wl[...], preferred_element_type=jnp.float32)
         + jnp.dot(x_ref[...], wr[...], preferred_element_type=jnp.float32)
         + b[...])
    row = pl.program_id(0) * BLK + lax.broadcasted_iota(jnp.int32, (BLK, 1), 0)
    h_ref[...] = jnp.where(row < N, jnp.maximum(h, 0.0), 0.0)
    inv_ref[...] = inv


_dense1 = pl.pallas_call(
    _dense1_body,
    grid=(N_PAD // BLK,),
    in_specs=[
        pl.BlockSpec((BLK, D_AUG), lambda i: (i, 0)),
        pl.BlockSpec((BLK, D_AUG), lambda i: (i, 0)),
        pl.BlockSpec((BLK, D), lambda i: (i, 0)),
        pl.BlockSpec((D, D), lambda i: (0, 0)),
        pl.BlockSpec((D, D), lambda i: (0, 0)),
        pl.BlockSpec((1, D), lambda i: (0, 0)),
    ],
    out_specs=[pl.BlockSpec((BLK, D), lambda i: (i, 0)),
               pl.BlockSpec((BLK, 1), lambda i: (i, 0))],
    out_shape=[jax.ShapeDtypeStruct((N_PAD, D), jnp.float32),
               jax.ShapeDtypeStruct((N_PAD, 1), jnp.float32)],
)


def _dense2_body(p0, p1, h_ref, inv_ref, wl, wr, b, out_ref):
    mean = (p0[...] + p1[...]) * inv_ref[...]
    out_ref[...] = (jnp.dot(mean, wl[...], preferred_element_type=jnp.float32)
                    + jnp.dot(h_ref[...], wr[...], preferred_element_type=jnp.float32)
                    + b[...])


_dense2 = pl.pallas_call(
    _dense2_body,
    grid=(N_PAD // BLK,),
    in_specs=[
        pl.BlockSpec((BLK, D), lambda i: (i, 0)),
        pl.BlockSpec((BLK, D), lambda i: (i, 0)),
        pl.BlockSpec((BLK, D), lambda i: (i, 0)),
        pl.BlockSpec((BLK, 1), lambda i: (i, 0)),
        pl.BlockSpec((D, D), lambda i: (0, 0)),
        pl.BlockSpec((D, D), lambda i: (0, 0)),
        pl.BlockSpec((1, D), lambda i: (0, 0)),
    ],
    out_specs=pl.BlockSpec((BLK, D), lambda i: (i, 0)),
    out_shape=jax.ShapeDtypeStruct((N_PAD, D), jnp.float32),
)


def kernel(x, edge_index, W1l, b1l, W1r, W2l, b2l, W2r):
    src = edge_index[0].astype(jnp.int32)
    dst = edge_index[1].astype(jnp.int32)

    def split_idx(sega, segb):
        ea = 16 * 2 * sega * KA
        e_pad = ea + 16 * 2 * segb * KB
        pad_dst = jnp.arange(e_pad - E, dtype=jnp.int32) % N
        src_p = jnp.concatenate([src, jnp.full((e_pad - E,), N, jnp.int32)])
        dst_p = jnp.concatenate([dst, pad_dst])
        idxa = jnp.stack([src_p[:ea].reshape(16, 2, sega, KA),
                          dst_p[:ea].reshape(16, 2, sega, KA)], axis=2)
        idxb = jnp.stack([src_p[ea:].reshape(16, 2, segb, KB),
                          dst_p[ea:].reshape(16, 2, segb, KB)], axis=2)
        return idxa, idxb

    idxa1, idxb1 = split_idx(SEGA1, SEGB1)
    idxa2, idxb2 = split_idx(SEGA2, SEGB2)

    x_aug = jnp.zeros((N_PAD, D_AUG), jnp.float32)
    x_aug = x_aug.at[:N, :D].set(x)
    x_aug = x_aug.at[:N, D].set(1.0)
    x_pad = x_aug[:, :D]
    z_aug = jnp.zeros((STRIPE, D_AUG), jnp.float32)
    z_plain = jnp.zeros((STRIPE, D), jnp.float32)

    p = _agg_aug(x_aug, idxa1, idxb1, z_aug)
    h, inv = _dense1(p[:N_PAD], p[N_PAD:], x_pad, W1l.T, W1r.T, b1l[None, :])
    p2 = _agg_plain(h, idxa2, idxb2, z_plain)
    out = _dense2(p2[:N_PAD], p2[N_PAD:], h, inv, W2l.T, W2r.T, b2l[None, :])
    return out[:N]


# speedup vs baseline: 1.0614x; 1.0538x over previous
import jax
import jax.numpy as jnp
from jax import lax
from jax.experimental import pallas as pl
from jax.experimental.pallas import tpu as pltpu
from jax.experimental.pallas import tpu_sc as plsc

N = 10000
E = 320000
D = 128
N_PAD = 10240
ACC_ROWS = 10000
D_AUG = 144
NW = 32
KA = 64
KB = 128
SEGA1, SEGB1 = 108, 25
SEGA2, SEGB2 = 108, 25
STRIPE = ACC_ROWS // 16
BLK = 512


def _make_agg(d, nbuf, sega, segb):
    mesh = plsc.VectorSubcoreMesh(core_axis_name="c", subcore_axis_name="s")

    def body(x_hbm, idxa_hbm, idxb_hbm, z_hbm, out_hbm,
             srcva, dstva, srcvb, dstvb, buf, sems, acc):
        c = lax.axis_index("c")
        s = lax.axis_index("s")
        pltpu.sync_copy(z_hbm, acc.at[pl.ds(s * STRIPE, STRIPE)])
        plsc.subcore_barrier()

        bufs = [buf.at[pl.ds(b * KA, KA)] for b in range(nbuf)]

        @pl.when(c == 0)
        def _sc0():
            for seg in range(2):
                pltpu.sync_copy(idxa_hbm.at[s, seg, 0], srcva)
                pltpu.sync_copy(idxa_hbm.at[s, seg, 1], dstva)
                for b in range(nbuf):
                    pltpu.async_copy(x_hbm.at[srcva.at[b]], bufs[b], sems[b])

                def group(g, carry):
                    for b in range(nbuf):
                        j = g * nbuf + b
                        pltpu.make_async_copy(x_hbm.at[srcva.at[j]], bufs[b],
                                              sems[b]).wait()
                        pltpu.sync_copy(bufs[b], acc.at[dstva.at[j]], add=True)
                        pltpu.async_copy(x_hbm.at[srcva.at[j + nbuf]],
                                         bufs[b], sems[b])
                    return carry

                lax.fori_loop(0, sega // nbuf - 1, group, 0)
                for b in range(nbuf):
                    j = sega - nbuf + b
                    pltpu.make_async_copy(x_hbm.at[srcva.at[j]], bufs[b],
                                          sems[b]).wait()
                    pltpu.sync_copy(bufs[b], acc.at[dstva.at[j]], add=True)

        @pl.when(c == 1)
        def _sc1():
            sbuf = buf.at[pl.ds(0, KB)]
            for seg in range(2):
                pltpu.sync_copy(idxb_hbm.at[s, seg, 0], srcvb)
                pltpu.sync_copy(idxb_hbm.at[s, seg, 1], dstvb)

                def step(j, carry):
                    pltpu.sync_copy(x_hbm.at[srcvb.at[j]], sbuf)
                    pltpu.sync_copy(sbuf, acc.at[dstvb.at[j]], add=True)
                    return carry

                lax.fori_loop(0, segb, step, 0)

        plsc.subcore_barrier()
        pltpu.sync_copy(acc.at[pl.ds(s * STRIPE, STRIPE)],
                        out_hbm.at[pl.ds(c * N_PAD + s * STRIPE, STRIPE)])

    return pl.kernel(
        body,
        out_type=jax.ShapeDtypeStruct((2 * N_PAD, d), jnp.float32),
        mesh=mesh,
        compiler_params=pltpu.CompilerParams(use_tc_tiling_on_sc=False),
        scratch_types=[
            pltpu.VMEM((sega, KA), jnp.int32),
            pltpu.VMEM((sega, KA), jnp.int32),
            pltpu.VMEM((segb, KB), jnp.int32),
            pltpu.VMEM((segb, KB), jnp.int32),
            pltpu.VMEM((nbuf * KA if nbuf * KA > KB else KB, d), jnp.float32),
            [pltpu.SemaphoreType.DMA for _ in range(nbuf)],
            pltpu.VMEM_SHARED((ACC_ROWS, d), jnp.float32),
        ],
    )


_agg_aug = _make_agg(D_AUG, 2, SEGA1, SEGB1)
_agg_plain = _make_agg(D, 3, SEGA2, SEGB2)


def _dense1_body(p0, p1, x_ref, wl, wr, b, h_ref, inv_ref):
    s = p0[...] + p1[...]
    deg = s[:, D:D + 1]
    inv = 1.0 / jnp.maximum(deg, 1.0)
    mean = s[:, :D] * inv
    h = (jnp.dot(mean, wl[...], preferred_element_type=jnp.float32)
         + jnp.dot(x_ref[...], wr[...], preferred_element_type=jnp.float32)
         + b[...])
    row = pl.program_id(0) * BLK + lax.broadcasted_iota(jnp.int32, (BLK, 1), 0)
    h_ref[...] = jnp.where(row < N, jnp.maximum(h, 0.0), 0.0)
    inv_ref[...] = inv


_dense1 = pl.pallas_call(
    _dense1_body,
    grid=(N_PAD // BLK,),
    in_specs=[
        pl.BlockSpec((BLK, D_AUG), lambda i: (i, 0)),
        pl.BlockSpec((BLK, D_AUG), lambda i: (i + N_PAD // BLK, 0)),
        pl.BlockSpec((BLK, D), lambda i: (i, 0)),
        pl.BlockSpec((D, D), lambda i: (0, 0)),
        pl.BlockSpec((D, D), lambda i: (0, 0)),
        pl.BlockSpec((1, D), lambda i: (0, 0)),
    ],
    out_specs=[pl.BlockSpec((BLK, D), lambda i: (i, 0)),
               pl.BlockSpec((BLK, 1), lambda i: (i, 0))],
    out_shape=[jax.ShapeDtypeStruct((N_PAD, D), jnp.float32),
               jax.ShapeDtypeStruct((N_PAD, 1), jnp.float32)],
)


def _dense2_body(p0, p1, h_ref, inv_ref, wl, wr, b, out_ref):
    mean = (p0[...] + p1[...]) * inv_ref[...]
    out_ref[...] = (jnp.dot(mean, wl[...], preferred_element_type=jnp.float32)
                    + jnp.dot(h_ref[...], wr[...], preferred_element_type=jnp.float32)
                    + b[...])


_dense2 = pl.pallas_call(
    _dense2_body,
    grid=(N_PAD // BLK,),
    in_specs=[
        pl.BlockSpec((BLK, D), lambda i: (i, 0)),
        pl.BlockSpec((BLK, D), lambda i: (i + N_PAD // BLK, 0)),
        pl.BlockSpec((BLK, D), lambda i: (i, 0)),
        pl.BlockSpec((BLK, 1), lambda i: (i, 0)),
        pl.BlockSpec((D, D), lambda i: (0, 0)),
        pl.BlockSpec((D, D), lambda i: (0, 0)),
        pl.BlockSpec((1, D), lambda i: (0, 0)),
    ],
    out_specs=pl.BlockSpec((BLK, D), lambda i: (i, 0)),
    out_shape=jax.ShapeDtypeStruct((N_PAD, D), jnp.float32),
)


def kernel(x, edge_index, W1l, b1l, W1r, W2l, b2l, W2r):
    src = edge_index[0].astype(jnp.int32)
    dst = edge_index[1].astype(jnp.int32)

    def split_idx(sega, segb):
        ea = 16 * 2 * sega * KA
        e_pad = ea + 16 * 2 * segb * KB
        pad_dst = jnp.arange(e_pad - E, dtype=jnp.int32) % N
        src_p = jnp.concatenate([src, jnp.full((e_pad - E,), N, jnp.int32)])
        dst_p = jnp.concatenate([dst, pad_dst])
        idxa = jnp.stack([src_p[:ea].reshape(16, 2, sega, KA),
                          dst_p[:ea].reshape(16, 2, sega, KA)], axis=2)
        idxb = jnp.stack([src_p[ea:].reshape(16, 2, segb, KB),
                          dst_p[ea:].reshape(16, 2, segb, KB)], axis=2)
        return idxa, idxb

    idxa1, idxb1 = split_idx(SEGA1, SEGB1)
    idxa2, idxb2 = idxa1, idxb1

    x_aug = jnp.zeros((N_PAD, D_AUG), jnp.float32)
    x_aug = x_aug.at[:N, :D].set(x)
    x_aug = x_aug.at[:N, D].set(1.0)
    z_aug = jnp.zeros((STRIPE, D_AUG), jnp.float32)
    z_plain = jnp.zeros((STRIPE, D), jnp.float32)

    p = _agg_aug(x_aug, idxa1, idxb1, z_aug)
    h, inv = _dense1(p, p, x_aug, W1l.T, W1r.T, b1l[None, :])
    p2 = _agg_plain(h, idxa2, idxb2, z_plain)
    out = _dense2(p2, p2, h, inv, W2l.T, W2r.T, b2l[None, :])
    return out[:N]
